# Initial kernel scaffold; baseline (speedup 1.0000x reference)
import jax
import jax.numpy as jnp
from jax.experimental import pallas as pl


def kernel(x, edge_index, W_rel1, b_rel1, W_root1, W_rel2, b_rel2, W_root2, W_out, b_out):
    raise NotImplementedError("write your pallas kernel here")


# speedup vs baseline: 3.5646x; 3.5646x over previous
import functools

import jax
import jax.numpy as jnp
from jax import lax
from jax.experimental import pallas as pl
from jax.experimental.pallas import tpu as pltpu
from jax.experimental.pallas import tpu_sc as plsc

N_NODES = 10000
D = 128
N_EDGES = 320000

NC = 2
NS = 16
NW = NC * NS
E_W = N_EDGES // NW
CHUNK = 128
E_PAD = 10240
NCH = E_PAD // CHUNK
N_PAD = 10240
ROWS_T = N_PAD // NS

_sc_mesh = plsc.VectorSubcoreMesh(core_axis_name="c", subcore_axis_name="s")


@functools.partial(
    pl.kernel,
    out_type=jax.ShapeDtypeStruct((NC, N_PAD, D), jnp.float32),
    mesh=_sc_mesh,
    scratch_types=[
        pltpu.VMEM((NCH, CHUNK), jnp.int32),
        pltpu.VMEM((NCH, CHUNK), jnp.int32),
        pltpu.VMEM((CHUNK, D), jnp.float32),
        pltpu.VMEM_SHARED((N_PAD, D), jnp.float32),
        pltpu.SemaphoreType.DMA,
        pltpu.SemaphoreType.DMA,
    ],
)
def _sc_segment_sum(x_hbm, src_hbm, dst_hbm, out_hbm,
                    src_v, dst_v, rows_v, agg_sh, sem0, sem1):
    cid = lax.axis_index("c")
    sid = lax.axis_index("s")

    pltpu.sync_copy(src_hbm.at[cid, sid], src_v)
    pltpu.sync_copy(dst_hbm.at[cid, sid], dst_v)

    zvec = jnp.zeros((16,), jnp.float32)

    def zstore(i, carry):
        rows_v[i // 8, pl.ds((i % 8) * 16, 16)] = zvec
        return carry

    lax.fori_loop(0, CHUNK * 8, zstore, 0, unroll=False)
    for r in range(ROWS_T // CHUNK):
        pltpu.sync_copy(rows_v,
                        agg_sh.at[pl.ds(sid * ROWS_T + r * CHUNK, CHUNK)])

    pltpu.async_copy(x_hbm.at[src_v.at[0]], rows_v, sem0)

    plsc.subcore_barrier()

    def chunk_step(c, _):
        pltpu.make_async_copy(x_hbm.at[src_v.at[c]], rows_v, sem0).wait()
        pltpu.sync_copy(rows_v, agg_sh.at[dst_v.at[c]], add=True)

        @pl.when(c + 1 < NCH)
        def _():
            pltpu.async_copy(x_hbm.at[src_v.at[c + 1]], rows_v, sem0)
        return _

    lax.fori_loop(0, NCH, lambda c, cr: (chunk_step(c, cr), 0)[1], 0,
                  unroll=False)

    plsc.subcore_barrier()

    pltpu.sync_copy(agg_sh.at[pl.ds(sid * ROWS_T, ROWS_T)],
                    out_hbm.at[cid, pl.ds(sid * ROWS_T, ROWS_T)])


BLK = 2000


def _dense1_body(agg_ref, x_ref, wrel_ref, b_ref, wroot_ref, out_ref):
    a = agg_ref[0] + agg_ref[1]
    h = (jnp.dot(a, wrel_ref[...], preferred_element_type=jnp.float32)
         + jnp.dot(x_ref[...], wroot_ref[...], preferred_element_type=jnp.float32)
         + b_ref[...])
    out_ref[...] = jnp.maximum(h, 0.0)


def _dense2_body(agg_ref, h_ref, wrel_ref, b_ref, wroot_ref, wout_ref,
                 bout_ref, out_ref):
    a = agg_ref[0] + agg_ref[1]
    h2 = (jnp.dot(a, wrel_ref[...], preferred_element_type=jnp.float32)
          + jnp.dot(h_ref[...], wroot_ref[...], preferred_element_type=jnp.float32)
          + b_ref[...])
    h2 = jnp.maximum(h2, 0.0)
    out_ref[...] = (jnp.sum(h2 * wout_ref[...], axis=1, keepdims=True)
                    + bout_ref[0, 0])


def _dense1(agg_pair, x, wrel_t, b, wroot_t):
    grid = (N_NODES // BLK,)
    return pl.pallas_call(
        _dense1_body,
        grid=grid,
        in_specs=[
            pl.BlockSpec((NC, BLK, D), lambda i: (0, i, 0)),
            pl.BlockSpec((BLK, D), lambda i: (i, 0)),
            pl.BlockSpec((D, D), lambda i: (0, 0)),
            pl.BlockSpec((1, D), lambda i: (0, 0)),
            pl.BlockSpec((D, D), lambda i: (0, 0)),
        ],
        out_specs=pl.BlockSpec((BLK, D), lambda i: (i, 0)),
        out_shape=jax.ShapeDtypeStruct((N_NODES, D), jnp.float32),
    )(agg_pair, x, wrel_t, b, wroot_t)


def _dense2(agg_pair, h, wrel_t, b, wroot_t, wout, bout):
    grid = (N_NODES // BLK,)
    return pl.pallas_call(
        _dense2_body,
        grid=grid,
        in_specs=[
            pl.BlockSpec((NC, BLK, D), lambda i: (0, i, 0)),
            pl.BlockSpec((BLK, D), lambda i: (i, 0)),
            pl.BlockSpec((D, D), lambda i: (0, 0)),
            pl.BlockSpec((1, D), lambda i: (0, 0)),
            pl.BlockSpec((D, D), lambda i: (0, 0)),
            pl.BlockSpec((1, D), lambda i: (0, 0)),
            pl.BlockSpec((1, 1), lambda i: (0, 0)),
        ],
        out_specs=pl.BlockSpec((BLK, 1), lambda i: (i, 0)),
        out_shape=jax.ShapeDtypeStruct((N_NODES, 1), jnp.float32),
    )(agg_pair, h, wrel_t, b, wroot_t, wout, bout)


def kernel(x, edge_index, W_rel1, b_rel1, W_root1, W_rel2, b_rel2, W_root2,
           W_out, b_out):
    ei = edge_index.astype(jnp.int32)
    pad_src = jnp.zeros((NW, E_PAD - E_W), jnp.int32)
    pad_dst = jnp.full((NW, E_PAD - E_W), N_NODES, jnp.int32)
    src = jnp.concatenate([ei[0].reshape(NW, E_W), pad_src], axis=1)
    dst = jnp.concatenate([ei[1].reshape(NW, E_W), pad_dst], axis=1)
    src = src.reshape(NC, NS, NCH, CHUNK)
    dst = dst.reshape(NC, NS, NCH, CHUNK)

    agg1 = _sc_segment_sum(x, src, dst)
    h1 = _dense1(agg1, x, W_rel1.T, b_rel1.reshape(1, D), W_root1.T)
    agg2 = _sc_segment_sum(h1, src, dst)
    out = _dense2(agg2, h1, W_rel2.T, b_rel2.reshape(1, D), W_root2.T,
                  W_out, b_out.reshape(1, 1))
    return out


# speedup vs baseline: 4.0994x; 1.1500x over previous
import functools

import jax
import jax.numpy as jnp
from jax import lax
from jax.experimental import pallas as pl
from jax.experimental.pallas import tpu as pltpu
from jax.experimental.pallas import tpu_sc as plsc

N_NODES = 10000
D = 128
N_EDGES = 320000

NC = 2
NS = 16
NW = NC * NS
E_W = N_EDGES // NW
CHUNK = 128
E_PAD = 10240
NCH = E_PAD // CHUNK
N_PAD = 10240
ROWS_T = N_PAD // NS

_sc_mesh = plsc.VectorSubcoreMesh(core_axis_name="c", subcore_axis_name="s")


@functools.partial(
    pl.kernel,
    out_type=jax.ShapeDtypeStruct((NC, N_PAD, D), jnp.float32),
    mesh=_sc_mesh,
    scratch_types=[
        pltpu.VMEM((4, 2, CHUNK), jnp.int32),
        pltpu.VMEM((2, CHUNK, D), jnp.float32),
        pltpu.VMEM_SHARED((N_PAD, D), jnp.float32),
        pltpu.SemaphoreType.DMA,
        pltpu.SemaphoreType.DMA,
        pltpu.SemaphoreType.DMA,
        pltpu.SemaphoreType.DMA,
        pltpu.SemaphoreType.DMA,
        pltpu.SemaphoreType.DMA,
    ],
)
def _sc_segment_sum(x_hbm, idx_hbm, out_hbm,
                    idx_v, rows_v, agg_sh,
                    sr0, sr1, si0, si1, si2, si3):
    cid = lax.axis_index("c")
    sid = lax.axis_index("s")
    sem_r = (sr0, sr1)
    sem_i = (si0, si1, si2, si3)

    zvec = jnp.zeros((16,), jnp.float32)

    def zstore(i, carry):
        rows_v[0, i // 8, pl.ds((i % 8) * 16, 16)] = zvec
        return carry

    lax.fori_loop(0, CHUNK * 8, zstore, 0, unroll=False)
    for r in range(ROWS_T // CHUNK):
        pltpu.sync_copy(rows_v.at[0],
                        agg_sh.at[pl.ds(sid * ROWS_T + r * CHUNK, CHUNK)])

    for q in range(4):
        pltpu.async_copy(idx_hbm.at[cid, sid, q], idx_v.at[q], sem_i[q])
    for b in range(2):
        pltpu.make_async_copy(idx_hbm.at[cid, sid, b], idx_v.at[b],
                              sem_i[b]).wait()
        pltpu.async_copy(x_hbm.at[idx_v.at[b, 0]], rows_v.at[b], sem_r[b])

    plsc.subcore_barrier()

    def chunk_step(c, q):
        b = q % 2
        pltpu.make_async_copy(x_hbm.at[idx_v.at[q, 0]], rows_v.at[b],
                              sem_r[b]).wait()
        pltpu.sync_copy(rows_v.at[b], agg_sh.at[idx_v.at[q, 1]], add=True)

        @pl.when(c + 4 < NCH)
        def _():
            pltpu.async_copy(idx_hbm.at[cid, sid, c + 4], idx_v.at[q],
                             sem_i[q])

        @pl.when(c + 2 < NCH)
        def _():
            q2 = (q + 2) % 4
            pltpu.make_async_copy(idx_hbm.at[cid, sid, c + 2], idx_v.at[q2],
                                  sem_i[q2]).wait()
            pltpu.async_copy(x_hbm.at[idx_v.at[q2, 0]], rows_v.at[b],
                             sem_r[b])

    def body(i, carry):
        g = 4 * i
        for q in range(4):
            chunk_step(g + q, q)
        return carry

    lax.fori_loop(0, NCH // 4, body, 0, unroll=False)

    plsc.subcore_barrier()

    pltpu.sync_copy(agg_sh.at[pl.ds(sid * ROWS_T, ROWS_T)],
                    out_hbm.at[cid, pl.ds(sid * ROWS_T, ROWS_T)])


BLK = 2000


def _dense1_body(agg_ref, x_ref, wrel_ref, b_ref, wroot_ref, out_ref):
    a = agg_ref[0] + agg_ref[1]
    h = (jnp.dot(a, wrel_ref[...], preferred_element_type=jnp.float32)
         + jnp.dot(x_ref[...], wroot_ref[...], preferred_element_type=jnp.float32)
         + b_ref[...])
    out_ref[...] = jnp.maximum(h, 0.0)


def _dense2_body(agg_ref, h_ref, wrel_ref, b_ref, wroot_ref, wout_ref,
                 bout_ref, out_ref):
    a = agg_ref[0] + agg_ref[1]
    h2 = (jnp.dot(a, wrel_ref[...], preferred_element_type=jnp.float32)
          + jnp.dot(h_ref[...], wroot_ref[...], preferred_element_type=jnp.float32)
          + b_ref[...])
    h2 = jnp.maximum(h2, 0.0)
    out_ref[...] = (jnp.sum(h2 * wout_ref[...], axis=1, keepdims=True)
                    + bout_ref[0, 0])


def _dense1(agg_pair, x, wrel_t, b, wroot_t):
    grid = (N_NODES // BLK,)
    return pl.pallas_call(
        _dense1_body,
        grid=grid,
        in_specs=[
            pl.BlockSpec((NC, BLK, D), lambda i: (0, i, 0)),
            pl.BlockSpec((BLK, D), lambda i: (i, 0)),
            pl.BlockSpec((D, D), lambda i: (0, 0)),
            pl.BlockSpec((1, D), lambda i: (0, 0)),
            pl.BlockSpec((D, D), lambda i: (0, 0)),
        ],
        out_specs=pl.BlockSpec((BLK, D), lambda i: (i, 0)),
        out_shape=jax.ShapeDtypeStruct((N_NODES, D), jnp.float32),
    )(agg_pair, x, wrel_t, b, wroot_t)


def _dense2(agg_pair, h, wrel_t, b, wroot_t, wout, bout):
    grid = (N_NODES // BLK,)
    return pl.pallas_call(
        _dense2_body,
        grid=grid,
        in_specs=[
            pl.BlockSpec((NC, BLK, D), lambda i: (0, i, 0)),
            pl.BlockSpec((BLK, D), lambda i: (i, 0)),
            pl.BlockSpec((D, D), lambda i: (0, 0)),
            pl.BlockSpec((1, D), lambda i: (0, 0)),
            pl.BlockSpec((D, D), lambda i: (0, 0)),
            pl.BlockSpec((1, D), lambda i: (0, 0)),
            pl.BlockSpec((1, 1), lambda i: (0, 0)),
        ],
        out_specs=pl.BlockSpec((BLK, 1), lambda i: (i, 0)),
        out_shape=jax.ShapeDtypeStruct((N_NODES, 1), jnp.float32),
    )(agg_pair, h, wrel_t, b, wroot_t, wout, bout)


def kernel(x, edge_index, W_rel1, b_rel1, W_root1, W_rel2, b_rel2, W_root2,
           W_out, b_out):
    ei = edge_index.astype(jnp.int32)
    pad_src = jnp.zeros((NW, E_PAD - E_W), jnp.int32)
    pad_dst = jnp.full((NW, E_PAD - E_W), N_NODES, jnp.int32)
    src = jnp.concatenate([ei[0].reshape(NW, E_W), pad_src], axis=1)
    dst = jnp.concatenate([ei[1].reshape(NW, E_W), pad_dst], axis=1)
    src = src.reshape(NC, NS, NCH, 1, CHUNK)
    dst = dst.reshape(NC, NS, NCH, 1, CHUNK)
    idx = jnp.concatenate([src, dst], axis=3)

    agg1 = _sc_segment_sum(x, idx)
    h1 = _dense1(agg1, x, W_rel1.T, b_rel1.reshape(1, D), W_root1.T)
    agg2 = _sc_segment_sum(h1, idx)
    out = _dense2(agg2, h1, W_rel2.T, b_rel2.reshape(1, D), W_root2.T,
                  W_out, b_out.reshape(1, 1))
    return out


# speedup vs baseline: 12.4859x; 3.0458x over previous
import functools

import jax
import jax.numpy as jnp
from jax import lax
from jax.experimental import pallas as pl
from jax.experimental.pallas import tpu as pltpu
from jax.experimental.pallas import tpu_sc as plsc

N_NODES = 10000
D = 128
N_EDGES = 320000

NC = 2
NS = 16
NW = NC * NS
E_W = N_EDGES // NW
CHUNK = 128
E_PAD = 10240
NCH = E_PAD // CHUNK
N_PAD = 10240
ROWS_T = N_PAD // NS

_sc_mesh = plsc.VectorSubcoreMesh(core_axis_name="c", subcore_axis_name="s")


@functools.partial(
    pl.kernel,
    out_type=jax.ShapeDtypeStruct((NC, N_PAD, D), jnp.float32),
    mesh=_sc_mesh,
    scratch_types=[
        pltpu.VMEM((4, 2, CHUNK), jnp.int32),
        pltpu.VMEM((2, CHUNK, D), jnp.float32),
        pltpu.VMEM_SHARED((N_PAD, D), jnp.float32),
        pltpu.SemaphoreType.DMA,
        pltpu.SemaphoreType.DMA,
        pltpu.SemaphoreType.DMA,
        pltpu.SemaphoreType.DMA,
        pltpu.SemaphoreType.DMA,
        pltpu.SemaphoreType.DMA,
    ],
)
def _sc_segment_sum(x_hbm, idx_hbm, out_hbm,
                    idx_v, rows_v, agg_sh,
                    sr0, sr1, si0, si1, si2, si3):
    cid = lax.axis_index("c")
    sid = lax.axis_index("s")
    sem_r = (sr0, sr1)
    sem_i = (si0, si1, si2, si3)

    zvec = jnp.zeros((16,), jnp.float32)

    def zstore(i, carry):
        rows_v[0, i // 8, pl.ds((i % 8) * 16, 16)] = zvec
        return carry

    lax.fori_loop(0, CHUNK * 8, zstore, 0, unroll=False)
    for r in range(ROWS_T // CHUNK):
        pltpu.sync_copy(rows_v.at[0],
                        agg_sh.at[pl.ds(sid * ROWS_T + r * CHUNK, CHUNK)])

    for q in range(4):
        pltpu.async_copy(idx_hbm.at[cid, sid, q], idx_v.at[q], sem_i[q])
    for b in range(2):
        pltpu.make_async_copy(idx_hbm.at[cid, sid, b], idx_v.at[b],
                              sem_i[b]).wait()
        pltpu.async_copy(x_hbm.at[idx_v.at[b, 0]], rows_v.at[b], sem_r[b])

    plsc.subcore_barrier()

    def chunk_step(c, q):
        b = q % 2
        pltpu.make_async_copy(x_hbm.at[idx_v.at[q, 0]], rows_v.at[b],
                              sem_r[b]).wait()
        pltpu.sync_copy(rows_v.at[b], agg_sh.at[idx_v.at[q, 1]], add=True)

        @pl.when(c + 4 < NCH)
        def _():
            pltpu.async_copy(idx_hbm.at[cid, sid, c + 4], idx_v.at[q],
                             sem_i[q])

        @pl.when(c + 2 < NCH)
        def _():
            q2 = (q + 2) % 4
            pltpu.make_async_copy(idx_hbm.at[cid, sid, c + 2], idx_v.at[q2],
                                  sem_i[q2]).wait()
            pltpu.async_copy(x_hbm.at[idx_v.at[q2, 0]], rows_v.at[b],
                             sem_r[b])

    def body(i, carry):
        g = 4 * i
        for q in range(4):
            chunk_step(g + q, q)
        return carry

    lax.fori_loop(0, NCH // 4, body, 0, unroll=False)

    plsc.subcore_barrier()

    pltpu.sync_copy(agg_sh.at[pl.ds(sid * ROWS_T, ROWS_T)],
                    out_hbm.at[cid, pl.ds(sid * ROWS_T, ROWS_T)])


BLK = 2000


def _dense1_body(agg_ref, x_ref, wrel_ref, b_ref, wroot_ref, out_ref):
    a = agg_ref[0] + agg_ref[1]
    h = (jnp.dot(a, wrel_ref[...], preferred_element_type=jnp.float32)
         + jnp.dot(x_ref[...], wroot_ref[...], preferred_element_type=jnp.float32)
         + b_ref[...])
    out_ref[...] = jnp.maximum(h, 0.0)


def _dense2_body(agg_ref, h_ref, wrel_ref, b_ref, wroot_ref, wout_ref,
                 bout_ref, out_ref):
    a = agg_ref[0] + agg_ref[1]
    h2 = (jnp.dot(a, wrel_ref[...], preferred_element_type=jnp.float32)
          + jnp.dot(h_ref[...], wroot_ref[...], preferred_element_type=jnp.float32)
          + b_ref[...])
    h2 = jnp.maximum(h2, 0.0)
    out_ref[...] = (jnp.sum(h2 * wout_ref[...], axis=1, keepdims=True)
                    + bout_ref[0, 0])


def _dense1(agg_pair, x, wrel_t, b, wroot_t):
    grid = (N_NODES // BLK,)
    return pl.pallas_call(
        _dense1_body,
        grid=grid,
        in_specs=[
            pl.BlockSpec((NC, BLK, D), lambda i: (0, i, 0)),
            pl.BlockSpec((BLK, D), lambda i: (i, 0)),
            pl.BlockSpec((D, D), lambda i: (0, 0)),
            pl.BlockSpec((1, D), lambda i: (0, 0)),
            pl.BlockSpec((D, D), lambda i: (0, 0)),
        ],
        out_specs=pl.BlockSpec((BLK, D), lambda i: (i, 0)),
        out_shape=jax.ShapeDtypeStruct((N_NODES, D), jnp.float32),
    )(agg_pair, x, wrel_t, b, wroot_t)


def _dense2(agg_pair, h, wrel_t, b, wroot_t, wout, bout):
    grid = (N_NODES // BLK,)
    return pl.pallas_call(
        _dense2_body,
        grid=grid,
        in_specs=[
            pl.BlockSpec((NC, BLK, D), lambda i: (0, i, 0)),
            pl.BlockSpec((BLK, D), lambda i: (i, 0)),
            pl.BlockSpec((D, D), lambda i: (0, 0)),
            pl.BlockSpec((1, D), lambda i: (0, 0)),
            pl.BlockSpec((D, D), lambda i: (0, 0)),
            pl.BlockSpec((1, D), lambda i: (0, 0)),
            pl.BlockSpec((1, 1), lambda i: (0, 0)),
        ],
        out_specs=pl.BlockSpec((BLK, 1), lambda i: (i, 0)),
        out_shape=jax.ShapeDtypeStruct((N_NODES, 1), jnp.float32),
    )(agg_pair, h, wrel_t, b, wroot_t, wout, bout)


def kernel(x, edge_index, W_rel1, b_rel1, W_root1, W_rel2, b_rel2, W_root2,
           W_out, b_out):
    ei = edge_index.astype(jnp.int32)
    k = jnp.arange(E_PAD - E_W, dtype=jnp.int32)[None, :]
    w = jnp.arange(NW, dtype=jnp.int32)[:, None]
    pad_src = (w * 977 + k * 41) % N_NODES
    pad_dst = N_NODES + (w + k) % (N_PAD - N_NODES)
    src = jnp.concatenate([ei[0].reshape(NW, E_W), pad_src], axis=1)
    dst = jnp.concatenate([ei[1].reshape(NW, E_W), pad_dst], axis=1)
    src = src.reshape(NC, NS, NCH, 1, CHUNK)
    dst = dst.reshape(NC, NS, NCH, 1, CHUNK)
    idx = jnp.concatenate([src, dst], axis=3)

    agg1 = _sc_segment_sum(x, idx)
    h1 = _dense1(agg1, x, W_rel1.T, b_rel1.reshape(1, D), W_root1.T)
    agg2 = _sc_segment_sum(h1, idx)
    out = _dense2(agg2, h1, W_rel2.T, b_rel2.reshape(1, D), W_root2.T,
                  W_out, b_out.reshape(1, 1))
    return out


# speedup vs baseline: 13.5328x; 1.0838x over previous
import functools

import jax
import jax.numpy as jnp
from jax import lax
from jax.experimental import pallas as pl
from jax.experimental.pallas import tpu as pltpu
from jax.experimental.pallas import tpu_sc as plsc

N_NODES = 10000
D = 128
N_EDGES = 320000

NC = 2
NS = 16
NW = NC * NS
E_W = N_EDGES // NW
CHUNK = 80
E_PAD = 10240
NCH = E_PAD // CHUNK
N_PAD = 10240
ROWS_T = N_PAD // NS

_sc_mesh = plsc.VectorSubcoreMesh(core_axis_name="c", subcore_axis_name="s")


@functools.partial(
    pl.kernel,
    out_type=jax.ShapeDtypeStruct((NC, N_PAD, D), jnp.float32),
    mesh=_sc_mesh,
    scratch_types=[
        pltpu.VMEM((4, 2, CHUNK), jnp.int32),
        pltpu.VMEM((4, CHUNK, D), jnp.float32),
        pltpu.VMEM_SHARED((N_PAD, D), jnp.float32),
        pltpu.SemaphoreType.DMA,
        pltpu.SemaphoreType.DMA,
        pltpu.SemaphoreType.DMA,
        pltpu.SemaphoreType.DMA,
        pltpu.SemaphoreType.DMA,
        pltpu.SemaphoreType.DMA,
        pltpu.SemaphoreType.DMA,
        pltpu.SemaphoreType.DMA,
    ],
)
def _sc_segment_sum(x_hbm, idx_hbm, out_hbm,
                    idx_v, rows_v, agg_sh,
                    sr0, sr1, sr2, sr3, si0, si1, si2, si3):
    cid = lax.axis_index("c")
    sid = lax.axis_index("s")
    sem_r = (sr0, sr1, sr2, sr3)
    sem_i = (si0, si1, si2, si3)

    zvec = jnp.zeros((16,), jnp.float32)

    def zstore(i, carry):
        rows_v[0, i // 8, pl.ds((i % 8) * 16, 16)] = zvec
        return carry

    lax.fori_loop(0, CHUNK * 8, zstore, 0, unroll=False)
    for r in range(ROWS_T // CHUNK):
        pltpu.sync_copy(rows_v.at[0],
                        agg_sh.at[pl.ds(sid * ROWS_T + r * CHUNK, CHUNK)])

    for q in range(4):
        pltpu.async_copy(idx_hbm.at[cid, sid, q], idx_v.at[q], sem_i[q])
    for b in range(3):
        pltpu.make_async_copy(idx_hbm.at[cid, sid, b], idx_v.at[b],
                              sem_i[b]).wait()
        pltpu.async_copy(x_hbm.at[idx_v.at[b, 0]], rows_v.at[b], sem_r[b])

    plsc.subcore_barrier()

    def chunk_step(c, q):
        pltpu.make_async_copy(x_hbm.at[idx_v.at[q, 0]], rows_v.at[q],
                              sem_r[q]).wait()

        pltpu.sync_copy(rows_v.at[q], agg_sh.at[idx_v.at[q, 1]], add=True)

        @pl.when(c + 4 < NCH)
        def _():
            pltpu.async_copy(idx_hbm.at[cid, sid, c + 4], idx_v.at[q],
                             sem_i[q])

        @pl.when(c + 3 < NCH)
        def _():
            q3 = (q + 3) % 4
            pltpu.make_async_copy(idx_hbm.at[cid, sid, c + 3], idx_v.at[q3],
                                  sem_i[q3]).wait()
            pltpu.async_copy(x_hbm.at[idx_v.at[q3, 0]], rows_v.at[q3],
                             sem_r[q3])

    def body(i, carry):
        g = 4 * i
        for q in range(4):
            chunk_step(g + q, q)
        return carry

    lax.fori_loop(0, NCH // 4, body, 0, unroll=False)

    plsc.subcore_barrier()

    pltpu.sync_copy(agg_sh.at[pl.ds(sid * ROWS_T, ROWS_T)],
                    out_hbm.at[cid, pl.ds(sid * ROWS_T, ROWS_T)])


BLK = 2000


def _dense1_body(agg_ref, x_ref, wrel_ref, b_ref, wroot_ref, out_ref):
    a = agg_ref[0] + agg_ref[1]
    h = (jnp.dot(a, wrel_ref[...], preferred_element_type=jnp.float32)
         + jnp.dot(x_ref[...], wroot_ref[...], preferred_element_type=jnp.float32)
         + b_ref[...])
    out_ref[...] = jnp.maximum(h, 0.0)


def _dense2_body(agg_ref, h_ref, wrel_ref, b_ref, wroot_ref, wout_ref,
                 bout_ref, out_ref):
    a = agg_ref[0] + agg_ref[1]
    h2 = (jnp.dot(a, wrel_ref[...], preferred_element_type=jnp.float32)
          + jnp.dot(h_ref[...], wroot_ref[...], preferred_element_type=jnp.float32)
          + b_ref[...])
    h2 = jnp.maximum(h2, 0.0)
    out_ref[...] = (jnp.sum(h2 * wout_ref[...], axis=1, keepdims=True)
                    + bout_ref[0, 0])


def _dense1(agg_pair, x, wrel_t, b, wroot_t):
    grid = (N_NODES // BLK,)
    return pl.pallas_call(
        _dense1_body,
        grid=grid,
        in_specs=[
            pl.BlockSpec((NC, BLK, D), lambda i: (0, i, 0)),
            pl.BlockSpec((BLK, D), lambda i: (i, 0)),
            pl.BlockSpec((D, D), lambda i: (0, 0)),
            pl.BlockSpec((1, D), lambda i: (0, 0)),
            pl.BlockSpec((D, D), lambda i: (0, 0)),
        ],
        out_specs=pl.BlockSpec((BLK, D), lambda i: (i, 0)),
        out_shape=jax.ShapeDtypeStruct((N_NODES, D), jnp.float32),
    )(agg_pair, x, wrel_t, b, wroot_t)


def _dense2(agg_pair, h, wrel_t, b, wroot_t, wout, bout):
    grid = (N_NODES // BLK,)
    return pl.pallas_call(
        _dense2_body,
        grid=grid,
        in_specs=[
            pl.BlockSpec((NC, BLK, D), lambda i: (0, i, 0)),
            pl.BlockSpec((BLK, D), lambda i: (i, 0)),
            pl.BlockSpec((D, D), lambda i: (0, 0)),
            pl.BlockSpec((1, D), lambda i: (0, 0)),
            pl.BlockSpec((D, D), lambda i: (0, 0)),
            pl.BlockSpec((1, D), lambda i: (0, 0)),
            pl.BlockSpec((1, 1), lambda i: (0, 0)),
        ],
        out_specs=pl.BlockSpec((BLK, 1), lambda i: (i, 0)),
        out_shape=jax.ShapeDtypeStruct((N_NODES, 1), jnp.float32),
    )(agg_pair, h, wrel_t, b, wroot_t, wout, bout)


def kernel(x, edge_index, W_rel1, b_rel1, W_root1, W_rel2, b_rel2, W_root2,
           W_out, b_out):
    ei = edge_index.astype(jnp.int32)
    k = jnp.arange(E_PAD - E_W, dtype=jnp.int32)[None, :]
    w = jnp.arange(NW, dtype=jnp.int32)[:, None]
    pad_src = (w * 977 + k * 41) % N_NODES
    pad_dst = N_NODES + (w + k) % (N_PAD - N_NODES)
    src = jnp.concatenate([ei[0].reshape(NW, E_W), pad_src], axis=1)
    dst = jnp.concatenate([ei[1].reshape(NW, E_W), pad_dst], axis=1)
    src = src.reshape(NC, NS, NCH, 1, CHUNK)
    dst = dst.reshape(NC, NS, NCH, 1, CHUNK)
    idx = jnp.concatenate([src, dst], axis=3)

    agg1 = _sc_segment_sum(x, idx)
    h1 = _dense1(agg1, x, W_rel1.T, b_rel1.reshape(1, D), W_root1.T)
    agg2 = _sc_segment_sum(h1, idx)
    out = _dense2(agg2, h1, W_rel2.T, b_rel2.reshape(1, D), W_root2.T,
                  W_out, b_out.reshape(1, 1))
    return out
